# baseline (speedup 1.0000x reference)
import functools

import jax
import jax.numpy as jnp
from jax import lax
from jax.experimental import pallas as pl
from jax.experimental.pallas import tpu as pltpu
from jax.experimental.pallas import tpu_sc as plsc

NC = 2
NS = 16
NT = NC * NS
CG = 4
IB = 128
NSUB = 10
CB = IB * NSUB
G = 64


def _sc_mesh():
    return plsc.VectorSubcoreMesh(
        core_axis_name="c", subcore_axis_name="s", num_cores=NC, num_subcores=NS
    )


def _deg_call(col, n_pad):
    e = col.shape[0]
    epw = e // NT
    dcb = 10000
    nch = epw // dcb
    nv = n_pad // 16

    @functools.partial(
        pl.kernel,
        out_type=jax.ShapeDtypeStruct((NT, n_pad), jnp.float32),
        mesh=_sc_mesh(),
        compiler_params=pltpu.CompilerParams(needs_layout_passes=False),
        scratch_types=[
            pltpu.VMEM((dcb,), jnp.int32),
            pltpu.VMEM((n_pad,), jnp.float32),
        ],
    )
    def deg_kernel(col_hbm, out_hbm, colb_v, table_v):
        c = lax.axis_index("c")
        s = lax.axis_index("s")
        wid = s * NC + c

        def zfill(i, _):
            table_v[pl.ds(i * 16, 16)] = jnp.zeros((16,), jnp.float32)
            return 0

        lax.fori_loop(0, nv, zfill, 0)

        ones16 = jnp.full((16,), 1.0, jnp.float32)

        def chunk(k, _):
            base = wid * epw + k * dcb
            pltpu.sync_copy(col_hbm.at[pl.ds(base, dcb)], colb_v)

            @plsc.parallel_loop(0, dcb // 16, unroll=4)
            def grp(j):
                idx = colb_v[pl.ds(j * 16, 16)]
                plsc.addupdate_scatter(table_v, [idx], ones16)

            return 0

        lax.fori_loop(0, nch, chunk, 0)
        pltpu.sync_copy(table_v, out_hbm.at[wid])

    return deg_kernel(col)


def _scatter_call(gf, row, col, n_pad):
    e = row.shape[0]
    scb = 10000
    nch = e // scb
    accn = n_pad * CG

    @functools.partial(
        pl.kernel,
        out_type=jax.ShapeDtypeStruct((NT, accn), jnp.float32),
        mesh=_sc_mesh(),
        compiler_params=pltpu.CompilerParams(needs_layout_passes=False),
        scratch_types=[
            pltpu.VMEM((scb,), jnp.int32),
            pltpu.VMEM((scb,), jnp.int32),
            pltpu.VMEM((accn,), jnp.float32),
            pltpu.VMEM((accn,), jnp.float32),
        ],
    )
    def scatter_kernel(gf_hbm, row_hbm, col_hbm, out_hbm,
                       rowb_v, colb_v, gtab_v, acc_v):
        c = lax.axis_index("c")
        s = lax.axis_index("s")
        wid = s * NC + c

        pltpu.sync_copy(gf_hbm.at[pl.ds(wid * accn, accn)], gtab_v)

        def zfill(i, _):
            acc_v[pl.ds(i * 16, 16)] = jnp.zeros((16,), jnp.float32)
            return 0

        lax.fori_loop(0, accn // 16, zfill, 0)

        def chunk(k, _):
            pltpu.sync_copy(row_hbm.at[pl.ds(k * scb, scb)], rowb_v)
            pltpu.sync_copy(col_hbm.at[pl.ds(k * scb, scb)], colb_v)

            @plsc.parallel_loop(0, scb // 16, unroll=4)
            def grp(j):
                rowv = rowb_v[pl.ds(j * 16, 16)]
                colv = colb_v[pl.ds(j * 16, 16)]
                row4 = rowv * CG
                col4 = colv * CG
                for ch in range(CG):
                    vals = plsc.load_gather(gtab_v, [row4 + ch])
                    plsc.addupdate_scatter(acc_v, [col4 + ch], vals)

            return 0

        lax.fori_loop(0, nch, chunk, 0)
        pltpu.sync_copy(acc_v, out_hbm.at[wid])

    return scatter_kernel(gf, row, col)


_BLK = 512


def _mm1_call(x_p, w1, degp):
    n_pad, d = x_p.shape
    grid = n_pad // _BLK

    def body(x_ref, w_ref, dp_ref, g1_ref, dis_ref):
        ones = jnp.ones((NT, 1), jnp.float32)
        dn = (((0,), (0,)), ((), ()))
        deg = lax.dot_general(
            dp_ref[...], ones, dn, preferred_element_type=jnp.float32
        ) + 1.0
        dis = lax.rsqrt(deg)
        h = jnp.dot(x_ref[...], w_ref[...], preferred_element_type=jnp.float32)
        g1_ref[...] = h * dis
        dis_ref[...] = dis

    return pl.pallas_call(
        body,
        grid=(grid,),
        in_specs=[
            pl.BlockSpec((_BLK, d), lambda i: (i, 0)),
            pl.BlockSpec((d, d), lambda i: (0, 0)),
            pl.BlockSpec((NT, _BLK), lambda i: (0, i)),
        ],
        out_specs=[
            pl.BlockSpec((_BLK, d), lambda i: (i, 0)),
            pl.BlockSpec((_BLK, 1), lambda i: (i, 0)),
        ],
        out_shape=[
            jax.ShapeDtypeStruct((n_pad, d), jnp.float32),
            jax.ShapeDtypeStruct((n_pad, 1), jnp.float32),
        ],
    )(x_p, w1, degp)


def _mm2_call(g1, acc1, dis_col, w2, b1):
    n_pad, d = g1.shape
    grid = n_pad // _BLK

    def body(g1_ref, a_ref, dis_ref, w_ref, b_ref, g2_ref):
        a = g1_ref[...] + a_ref[...]
        t = jnp.maximum(dis_ref[...] * a + b_ref[...], 0.0)
        g2_ref[...] = jnp.dot(
            t, w_ref[...], preferred_element_type=jnp.float32
        ) * dis_ref[...]

    return pl.pallas_call(
        body,
        grid=(grid,),
        in_specs=[
            pl.BlockSpec((_BLK, d), lambda i: (i, 0)),
            pl.BlockSpec((_BLK, d), lambda i: (i, 0)),
            pl.BlockSpec((_BLK, 1), lambda i: (i, 0)),
            pl.BlockSpec((d, d), lambda i: (0, 0)),
            pl.BlockSpec((1, d), lambda i: (0, 0)),
        ],
        out_specs=pl.BlockSpec((_BLK, d), lambda i: (i, 0)),
        out_shape=jax.ShapeDtypeStruct((n_pad, d), jnp.float32),
    )(g1, acc1, dis_col, w2, b1)


def _final_call(g2, acc2, dis_col, batch_col, b2, fcw, fcb):
    n_pad, d = g2.shape
    dout = fcw.shape[1]
    grid = n_pad // _BLK

    def body(g2_ref, a_ref, dis_ref, bt_ref, b_ref, fcw_ref, fcb_ref,
             out_ref, sums_ref, cnt_ref):
        i = pl.program_id(0)

        @pl.when(i == 0)
        def _():
            sums_ref[...] = jnp.zeros_like(sums_ref)
            cnt_ref[...] = jnp.zeros_like(cnt_ref)

        a = g2_ref[...] + a_ref[...]
        r = jnp.maximum(dis_ref[...] * a + b_ref[...], 0.0)
        bt = bt_ref[...]
        p = (bt == lax.broadcasted_iota(jnp.int32, (_BLK, G), 1)).astype(
            jnp.float32
        )
        dn = (((0,), (0,)), ((), ()))
        sums_ref[...] += lax.dot_general(
            p, r, dn, preferred_element_type=jnp.float32
        )
        cnt_ref[...] += lax.dot_general(
            p, jnp.ones((_BLK, 1), jnp.float32), dn,
            preferred_element_type=jnp.float32,
        )

        @pl.when(i == grid - 1)
        def _():
            pooled = sums_ref[...] / jnp.maximum(cnt_ref[...], 1.0)
            out_ref[...] = jnp.dot(
                pooled, fcw_ref[...], preferred_element_type=jnp.float32
            ) + fcb_ref[...]

    return pl.pallas_call(
        body,
        grid=(grid,),
        in_specs=[
            pl.BlockSpec((_BLK, d), lambda i: (i, 0)),
            pl.BlockSpec((_BLK, d), lambda i: (i, 0)),
            pl.BlockSpec((_BLK, 1), lambda i: (i, 0)),
            pl.BlockSpec((_BLK, 1), lambda i: (i, 0)),
            pl.BlockSpec((1, d), lambda i: (0, 0)),
            pl.BlockSpec((d, dout), lambda i: (0, 0)),
            pl.BlockSpec((1, dout), lambda i: (0, 0)),
        ],
        out_specs=pl.BlockSpec((G, dout), lambda i: (0, 0)),
        out_shape=jax.ShapeDtypeStruct((G, dout), jnp.float32),
        scratch_shapes=[
            pltpu.VMEM((G, d), jnp.float32),
            pltpu.VMEM((G, 1), jnp.float32),
        ],
    )(g2, acc2, dis_col, batch_col, b2, fcw, fcb)


def _to_groups(g, n_pad):
    return g.reshape(n_pad, NT, CG).transpose(1, 0, 2).reshape(NT * n_pad * CG)


def _from_groups(acc, n_pad):
    return acc.reshape(NT, n_pad, CG).transpose(1, 0, 2).reshape(n_pad, NT * CG)


def kernel(x, edge_index, batch, W1, b1, W2, b2, fcW, fcb):
    n, d = x.shape
    n_pad = ((n + 2047) // 2048) * 2048

    row = edge_index[0].astype(jnp.int32)
    col = edge_index[1].astype(jnp.int32)
    x_p = jnp.pad(x, ((0, n_pad - n), (0, 0)))
    batch_col = jnp.pad(batch, (0, n_pad - n), constant_values=G).astype(
        jnp.int32
    )[:, None]

    degp = _deg_call(col, n_pad)
    g1, dis_col = _mm1_call(x_p, W1, degp)
    acc1 = _scatter_call(_to_groups(g1, n_pad), row, col, n_pad)
    g2 = _mm2_call(g1, _from_groups(acc1, n_pad), dis_col, W2, b1[None, :])
    acc2 = _scatter_call(_to_groups(g2, n_pad), row, col, n_pad)
    return _final_call(
        g2, _from_groups(acc2, n_pad), dis_col, batch_col, b2[None, :],
        fcW, fcb[None, :],
    )
